# baseline (speedup 1.0000x reference)
import functools

import jax
import jax.numpy as jnp
from jax import lax
from jax.experimental import pallas as pl
from jax.experimental.pallas import tpu as pltpu
from jax.experimental.pallas import tpu_sc as plsc

N_EMB = 32
LANES = 16
NUM_CORES = 2
NUM_SUBCORES = 16
NW = NUM_CORES * NUM_SUBCORES
IDX_CHUNK = 128


def _sc_body(b_per_w, n_chunks, users_hbm, movies_hbm, uemb_hbm, memb_hbm,
             out_hbm, uidx_v, midx_v, urows_v, mrows_v, out_v, sem):
    wid = lax.axis_index("s") * NUM_CORES + lax.axis_index("c")
    chunk_base = wid * n_chunks

    pltpu.sync_copy(users_hbm.at[pl.ds(chunk_base, n_chunks)], uidx_v)
    pltpu.sync_copy(movies_hbm.at[pl.ds(chunk_base, n_chunks)], midx_v)

    handles = []
    for c in range(n_chunks):
        dst = urows_v.at[pl.ds(c * IDX_CHUNK, IDX_CHUNK)]
        handles.append(pltpu.async_copy(uemb_hbm.at[uidx_v.at[c]], dst, sem))
        dst = mrows_v.at[pl.ds(c * IDX_CHUNK, IDX_CHUNK)]
        handles.append(pltpu.async_copy(memb_hbm.at[midx_v.at[c]], dst, sem))
    for h in handles:
        h.wait()

    lane = lax.iota(jnp.int32, LANES)

    def block(b, carry):
        row = b * LANES + lane
        acc = jnp.zeros((LANES,), jnp.float32)
        for j in range(N_EMB):
            col = jnp.bitwise_and(lane + j, N_EMB - 1)
            u = plsc.load_gather(urows_v, [row, col])
            m = plsc.load_gather(mrows_v, [row, col])
            acc = acc + u * m
        out_v[pl.ds(b * LANES, LANES)] = 0.5 + 4.5 / (1.0 + jnp.exp(-acc))
        return carry

    lax.fori_loop(0, b_per_w // LANES, block, 0)

    pltpu.sync_copy(out_v, out_hbm.at[pl.ds(wid * b_per_w, b_per_w)])


def kernel(users, movies, users_embedding, movies_embedding):
    batch = users.shape[0]
    assert batch % (NW * IDX_CHUNK) == 0
    b_per_w = batch // NW
    n_chunks = b_per_w // IDX_CHUNK

    users2d = users.astype(jnp.int32).reshape(batch // IDX_CHUNK, IDX_CHUNK)
    movies2d = movies.astype(jnp.int32).reshape(batch // IDX_CHUNK, IDX_CHUNK)

    mesh = plsc.VectorSubcoreMesh(core_axis_name="c", subcore_axis_name="s",
                                  num_cores=NUM_CORES, num_subcores=NUM_SUBCORES)
    fn = pl.kernel(
        functools.partial(_sc_body, b_per_w, n_chunks),
        out_type=jax.ShapeDtypeStruct((batch,), jnp.float32),
        mesh=mesh,
        scratch_types=[
            pltpu.VMEM((n_chunks, IDX_CHUNK), jnp.int32),
            pltpu.VMEM((n_chunks, IDX_CHUNK), jnp.int32),
            pltpu.VMEM((b_per_w, N_EMB), jnp.float32),
            pltpu.VMEM((b_per_w, N_EMB), jnp.float32),
            pltpu.VMEM((b_per_w,), jnp.float32),
            pltpu.SemaphoreType.DMA,
        ],
        compiler_params=pltpu.CompilerParams(needs_layout_passes=False,
                                             use_tc_tiling_on_sc=False),
    )
    return fn(users2d, movies2d, users_embedding, movies_embedding)


# speedup vs baseline: 1.4620x; 1.4620x over previous
import functools

import jax
import jax.numpy as jnp
from jax import lax
from jax.experimental import pallas as pl
from jax.experimental.pallas import tpu as pltpu
from jax.experimental.pallas import tpu_sc as plsc

N_EMB = 32
ROWS_PER_LINE = 4
LANES = 16
NUM_CORES = 2
NUM_SUBCORES = 16
NW = NUM_CORES * NUM_SUBCORES


def _sc_body(b_per_w, users_hbm, movies_hbm, uemb_hbm, memb_hbm,
             out_hbm, uidx_s, midx_s, urows_v, mrows_v, out_v, sem):
    wid = lax.axis_index("s") * NUM_CORES + lax.axis_index("c")
    base = wid * b_per_w
    n_groups = b_per_w // LANES

    pltpu.sync_copy(users_hbm.at[pl.ds(base, b_per_w)], uidx_s)
    pltpu.sync_copy(movies_hbm.at[pl.ds(base, b_per_w)], midx_s)

    def fire_chunk(g, carry):
        uvec = uidx_s[pl.ds(g * LANES, LANES)]
        mvec = midx_s[pl.ds(g * LANES, LANES)]
        handles = []
        for k in range(LANES):
            i = g * LANES + k
            line = lax.div(i, ROWS_PER_LINE)
            off = (k % ROWS_PER_LINE) * N_EMB
            dst = urows_v.at[line, pl.ds(off, N_EMB)]
            handles.append(pltpu.async_copy(uemb_hbm.at[uvec[k]], dst, sem))
            dst = mrows_v.at[line, pl.ds(off, N_EMB)]
            handles.append(pltpu.async_copy(memb_hbm.at[mvec[k]], dst, sem))
        for h in handles:
            h.wait()
        return carry

    lax.fori_loop(0, n_groups, fire_chunk, 0)

    lane = lax.iota(jnp.int32, LANES)

    def block(b, carry):
        row = b * LANES + lane
        line = lax.div(row, ROWS_PER_LINE)
        base_col = lax.rem(row, ROWS_PER_LINE) * N_EMB
        acc = jnp.zeros((LANES,), jnp.float32)
        for j in range(N_EMB):
            col = base_col + jnp.bitwise_and(lane + j, N_EMB - 1)
            u = plsc.load_gather(urows_v, [line, col])
            m = plsc.load_gather(mrows_v, [line, col])
            acc = acc + u * m
        out_v[pl.ds(b * LANES, LANES)] = 0.5 + 4.5 / (1.0 + jnp.exp(-acc))
        return carry

    lax.fori_loop(0, n_groups, block, 0)

    pltpu.sync_copy(out_v, out_hbm.at[pl.ds(base, b_per_w)])


def kernel(users, movies, users_embedding, movies_embedding):
    batch = users.shape[0]
    assert batch % (NW * LANES) == 0
    b_per_w = batch // NW

    mesh = plsc.VectorSubcoreMesh(core_axis_name="c", subcore_axis_name="s",
                                  num_cores=NUM_CORES, num_subcores=NUM_SUBCORES)
    fn = pl.kernel(
        functools.partial(_sc_body, b_per_w),
        out_type=jax.ShapeDtypeStruct((batch,), jnp.float32),
        mesh=mesh,
        scratch_types=[
            pltpu.VMEM((b_per_w,), jnp.int32),
            pltpu.VMEM((b_per_w,), jnp.int32),
            pltpu.VMEM((b_per_w // ROWS_PER_LINE, ROWS_PER_LINE * N_EMB), jnp.float32),
            pltpu.VMEM((b_per_w // ROWS_PER_LINE, ROWS_PER_LINE * N_EMB), jnp.float32),
            pltpu.VMEM((b_per_w,), jnp.float32),
            pltpu.SemaphoreType.DMA,
        ],
        compiler_params=pltpu.CompilerParams(needs_layout_passes=False),
    )
    return fn(users.astype(jnp.int32), movies.astype(jnp.int32),
              users_embedding, movies_embedding)
